# baseline (speedup 1.0000x reference)
import functools

import jax
import jax.numpy as jnp
import numpy as np
from jax import lax
from jax.experimental import pallas as pl
from jax.experimental.pallas import tpu as pltpu
from jax.experimental.pallas import tpu_sc as plsc

N = 10000
E = 320000
H = 128
NV = 16
DE = 16
EMB = 64
PE = 32
HID = 2 * H

F32 = jnp.float32


_BN = 1000


def _eigen_body(x_ref, vec_ref, val_ref, eps_ref, w1_ref, b1_ref, w2_ref,
                b2_ref, out_ref):
  ev = val_ref[...] + eps_ref[...]
  ev = jnp.where(jnp.isnan(ev), 0.0, ev)
  vec = jnp.where(jnp.isnan(vec_ref[...]), 0.0, vec_ref[...])
  w1 = w1_ref[...]
  b1 = b1_ref[...]
  w2 = w2_ref[...]
  b2 = b2_ref[...]
  w2b = w2.astype(jnp.bfloat16)
  acc = jnp.zeros((_BN, H), F32)
  for v in range(NV):
    t1 = vec[:, v:v + 1] * w1[0:1, :] + ev[:, v:v + 1] * w1[1:2, :] + b1
    t1 = jnp.maximum(t1, 0.0).astype(jnp.bfloat16)
    t2 = jnp.dot(t1, w2b, preferred_element_type=F32) + b2
    acc = acc + jnp.maximum(t2, 0.0)
  res = x_ref[...] + acc
  out_ref[0] = res[:, :H // 2]
  out_ref[1] = res[:, H // 2:]


def _eigen_stage(x, eigvecs, eigvals, eps_row, phi_W1, phi_b1, phi_W2,
                 phi_b2):
  grid = N // _BN
  return pl.pallas_call(
      _eigen_body,
      grid=(grid,),
      in_specs=[
          pl.BlockSpec((_BN, H), lambda i: (i, 0)),
          pl.BlockSpec((_BN, NV), lambda i: (i, 0)),
          pl.BlockSpec((_BN, NV), lambda i: (i, 0)),
          pl.BlockSpec((1, NV), lambda i: (0, 0)),
          pl.BlockSpec((2, HID), lambda i: (0, 0)),
          pl.BlockSpec((1, HID), lambda i: (0, 0)),
          pl.BlockSpec((HID, H), lambda i: (0, 0)),
          pl.BlockSpec((1, H), lambda i: (0, 0)),
      ],
      out_specs=pl.BlockSpec((2, _BN, H // 2), lambda i: (0, i, 0)),
      out_shape=jax.ShapeDtypeStruct((2, N, H // 2), F32),
  )(x, eigvecs, eigvals, eps_row, phi_W1, phi_b1, phi_W2, phi_b2)


_BE2 = 3200
_BE4 = 3200


def _dotT(a_ref, w_ref):
  return jax.lax.dot_general(a_ref[...], w_ref[...], (((0,), (0,)), ((), ())),
                             preferred_element_type=F32)


def _edge0_body(eaA_ref, eaB_ref, C0_ref, bL_ref, bR_ref, out_ref):
  EA = _dotT(eaA_ref, C0_ref)
  EB = _dotT(eaB_ref, C0_ref)
  hh = H // 2
  out_ref[0] = jnp.concatenate([EA[:, :hh], EB[:, :hh]], axis=1) + bL_ref[...]
  out_ref[1] = jnp.concatenate([EA[:, hh:], EB[:, hh:]], axis=1) + bR_ref[...]


def _edge0_stage(eaT, C0, bL, bR):
  grid = (E // 2) // _BE2
  nb = grid
  return pl.pallas_call(
      _edge0_body,
      grid=(grid,),
      in_specs=[
          pl.BlockSpec((DE, _BE2), lambda i: (0, i)),
          pl.BlockSpec((DE, _BE2), lambda i: (0, i + nb)),
          pl.BlockSpec((DE, H), lambda i: (0, 0)),
          pl.BlockSpec((1, H), lambda i: (0, 0)),
          pl.BlockSpec((1, H), lambda i: (0, 0)),
      ],
      out_specs=pl.BlockSpec((2, _BE2, H), lambda i: (0, i, 0)),
      out_shape=jax.ShapeDtypeStruct((2, E // 2, H), F32),
  )(eaT, eaT, C0, bL, bR)


def _edge1_body(t0_ref, t1_ref, t2_ref, t3_ref, C1_ref, bL_ref, bR_ref,
                out_ref):
  F = [_dotT(t, C1_ref) for t in (t0_ref, t1_ref, t2_ref, t3_ref)]
  qq = EMB // 2
  out_ref[0] = jnp.concatenate([f[:, :qq] for f in F], axis=1) + bL_ref[...]
  out_ref[1] = jnp.concatenate([f[:, qq:] for f in F], axis=1) + bR_ref[...]


def _edge1_stage(eaT, C1, bL, bR):
  grid = (E // 4) // _BE4
  nb = grid
  return pl.pallas_call(
      _edge1_body,
      grid=(grid,),
      in_specs=[
          pl.BlockSpec((DE, _BE4), lambda i: (0, i)),
          pl.BlockSpec((DE, _BE4), lambda i: (0, i + nb)),
          pl.BlockSpec((DE, _BE4), lambda i: (0, i + 2 * nb)),
          pl.BlockSpec((DE, _BE4), lambda i: (0, i + 3 * nb)),
          pl.BlockSpec((DE, EMB), lambda i: (0, 0)),
          pl.BlockSpec((1, 2 * EMB), lambda i: (0, 0)),
          pl.BlockSpec((1, 2 * EMB), lambda i: (0, 0)),
      ],
      out_specs=pl.BlockSpec((2, _BE4, 2 * EMB), lambda i: (0, i, 0)),
      out_shape=jax.ShapeDtypeStruct((2, E // 4, 2 * EMB), F32),
  )(eaT, eaT, eaT, eaT, C1, bL, bR)


_NC = 2
_NS = 16
_G = 80
_NP = 10240


def _make_sc_segment(D):
  DH = D // 2
  P = 128 // DH
  C = 200 * P
  Q = C // _G
  per_tile = E // _NS
  chunks = per_tile // C
  rp = _NP // _NS

  mesh = plsc.VectorSubcoreMesh(core_axis_name="c", subcore_axis_name="s",
                                num_cores=_NC, num_subcores=_NS)

  @functools.partial(
      pl.kernel,
      out_type=jax.ShapeDtypeStruct((_NP, D), F32),
      mesh=mesh,
      scratch_types=[
          pltpu.VMEM((2, P, C // P), jnp.int32),
          pltpu.VMEM((2, P, C // P), jnp.int32),
          pltpu.VMEM((2, Q * (_G // 16), 16), jnp.int32),
          pltpu.VMEM((2, Q, _G), jnp.int32),
          pltpu.VMEM((2, Q, _G), jnp.int32),
          pltpu.VMEM((C, DH), F32),
          pltpu.VMEM((C, DH), F32),
          pltpu.VMEM_SHARED((_NP, DH), F32),
          pltpu.SemaphoreType.DMA,
          pltpu.SemaphoreType.DMA,
          pltpu.SemaphoreType.DMA,
          pltpu.SemaphoreType.DMA,
          pltpu.SemaphoreType.DMA,
      ],
      compiler_params=pltpu.CompilerParams(use_tc_tiling_on_sc=False,
                                           needs_layout_passes=False),
  )
  def seg(h_hbm, e_hbm, srcP, dstP, itab_hbm, out_hbm, sraw, draw, itab,
          sidx, didx, bufA, bufB, agg, semLA, semLB, semGA, semGB, semS):
    c = lax.axis_index("c")
    s = lax.axis_index("s")
    bufs = (bufA, bufB)
    semL = (semLA, semLB)
    semG = (semGA, semGB)
    h_half = h_hbm.at[c]
    e_half = e_hbm.at[c]
    pltpu.sync_copy(itab_hbm, itab)

    def zrow(i, carry):
      for j in range(DH // 16):
        bufA[i, pl.ds(16 * j, 16)] = jnp.zeros((16,), F32)
      return carry

    lax.fori_loop(0, 128, zrow, 0)
    for q in range(rp // 128):
      pltpu.sync_copy(bufA.at[pl.ds(0, 128)],
                      agg.at[pl.ds(s * rp + q * 128, 128)])
    plsc.subcore_barrier()

    base_e = s * per_tile
    base_col = s * (per_tile // P)

    def fire_ld(k, p):
      col = base_col + k * (C // P)
      pltpu.async_copy(srcP.at[:, pl.ds(col, C // P)], sraw.at[p], semL[p])
      pltpu.async_copy(dstP.at[:, pl.ds(col, C // P)], draw.at[p], semL[p])
      pltpu.async_copy(e_half.at[pl.ds(base_e + k * C, C)], bufs[p], semL[p])

    def wait_ld(k, p):
      col = base_col + k * (C // P)
      pltpu.make_async_copy(srcP.at[:, pl.ds(col, C // P)], sraw.at[p],
                            semL[p]).wait()
      pltpu.make_async_copy(dstP.at[:, pl.ds(col, C // P)], draw.at[p],
                            semL[p]).wait()
      pltpu.make_async_copy(e_half.at[pl.ds(base_e + k * C, C)], bufs[p],
                            semL[p]).wait()
      for q in range(Q):
        for v in range(_G // 16):
          j = q * (_G // 16) + v
          i0 = itab[0, j]
          i1 = itab[1, j]
          sidx[p, q, pl.ds(16 * v, 16)] = plsc.load_gather(
              sraw.at[p], [i0, i1])
          didx[p, q, pl.ds(16 * v, 16)] = plsc.load_gather(
              draw.at[p], [i0, i1])

    def fire_gathers(p):
      for q in range(Q):
        pltpu.async_copy(h_half.at[sidx.at[p, q]],
                         bufs[p].at[pl.ds(q * _G, _G)], semG[p], add=True)

    def drain_gathers(p):
      for q in range(Q):
        pltpu.make_async_copy(h_half.at[sidx.at[p, q]],
                              bufs[p].at[pl.ds(q * _G, _G)],
                              semG[p]).wait()

    def fire_scatter(p):
      for q in range(Q):
        pltpu.async_copy(bufs[p].at[pl.ds(q * _G, _G)],
                         agg.at[didx.at[p, q]], semS, add=True)

    def drain_scatter(p):
      for q in range(Q):
        pltpu.make_async_copy(bufs[p].at[pl.ds(q * _G, _G)],
                              agg.at[didx.at[p, q]], semS).wait()

    def relu(p):
      buf = bufs[p]

      def rrow(i, cc):
        for u in range(4):
          for j in range(DH // 16):
            sl = pl.ds(16 * j, 16)
            buf[4 * i + u, sl] = jnp.maximum(buf[4 * i + u, sl], 0.0)
        return cc

      lax.fori_loop(0, C // 4, rrow, 0)

    def process(k, p):
      drain_gathers(p)
      relu(p)
      fire_scatter(p)

      @pl.when(k + 1 < chunks)
      def _():
        wait_ld(k + 1, 1 - p)
        fire_gathers(1 - p)

      drain_scatter(p)

      @pl.when(k + 2 < chunks)
      def _():
        fire_ld(k + 2, p)

    fire_ld(0, 0)
    wait_ld(0, 0)
    fire_gathers(0)
    fire_ld(1, 1)

    def pair(i, carry):
      k = 2 * i
      process(k, 0)

      @pl.when(k + 1 < chunks)
      def _():
        process(k + 1, 1)

      return carry

    lax.fori_loop(0, (chunks + 1) // 2, pair, 0)

    plsc.subcore_barrier()
    pltpu.sync_copy(agg.at[pl.ds(s * rp, rp)],
                    out_hbm.at[pl.ds(s * rp, rp), pl.ds(c * DH, DH)])

  return seg


def _destripe_table(D):
  DH = D // 2
  P = 128 // DH
  C = 200 * P
  Q = C // _G
  lanes = np.arange(16, dtype=np.int32)
  i0 = []
  i1 = []
  for q in range(Q):
    for v in range(_G // 16):
      off = (16 * v + _G * q) // P
      i0.append(lanes % P)
      i1.append(lanes // P + off)
  return jnp.asarray(np.stack([np.stack(i0), np.stack(i1)]), jnp.int32)


_sc_cache = {}


def _sc_segment(D):
  if D not in _sc_cache:
    _sc_cache[D] = _make_sc_segment(D)
  return _sc_cache[D]


def _node0_body(h_ref, agg_ref, sc_ref, w1_ref, b1_ref, w2_ref, b2_ref,
                out_ref):
  hv = jnp.concatenate([h_ref[0], h_ref[1]], axis=1)
  hv = sc_ref[0, 0] * hv + agg_ref[...]
  t = jnp.maximum(jnp.dot(hv, w1_ref[...], preferred_element_type=F32)
                  + b1_ref[...], 0.0)
  res = jnp.maximum(
      jnp.dot(t, w2_ref[...], preferred_element_type=F32) + b2_ref[...], 0.0)
  out_ref[0] = res[:, :EMB // 2]
  out_ref[1] = res[:, EMB // 2:]


def _node0_stage(hS, agg, scale, W1, b1_row, W2, b2_row):
  grid = N // _BN
  return pl.pallas_call(
      _node0_body,
      grid=(grid,),
      in_specs=[
          pl.BlockSpec((2, _BN, H // 2), lambda i: (0, i, 0)),
          pl.BlockSpec((_BN, H), lambda i: (i, 0)),
          pl.BlockSpec(memory_space=pltpu.SMEM),
          pl.BlockSpec((H, EMB), lambda i: (0, 0)),
          pl.BlockSpec((1, EMB), lambda i: (0, 0)),
          pl.BlockSpec((EMB, EMB), lambda i: (0, 0)),
          pl.BlockSpec((1, EMB), lambda i: (0, 0)),
      ],
      out_specs=pl.BlockSpec((2, _BN, EMB // 2), lambda i: (0, i, 0)),
      out_shape=jax.ShapeDtypeStruct((2, N, EMB // 2), F32),
  )(hS, agg, scale, W1, b1_row, W2, b2_row)


def _node1_body(h_ref, agg_ref, sc_ref, w1_ref, b1_ref, w2_ref, b2_ref,
                ow_ref, ob_ref, out_ref):
  hv = jnp.concatenate([h_ref[0], h_ref[1]], axis=1)
  hv = sc_ref[0, 0] * hv + agg_ref[...]
  t = jnp.maximum(jnp.dot(hv, w1_ref[...], preferred_element_type=F32)
                  + b1_ref[...], 0.0)
  t = jnp.maximum(jnp.dot(t, w2_ref[...], preferred_element_type=F32)
                  + b2_ref[...], 0.0)
  out_ref[...] = jnp.dot(t, ow_ref[...], preferred_element_type=F32) + ob_ref[...]


def _node1_stage(h1S, agg, scale, W1, b1_row, W2, b2_row, out_W, ob_row):
  grid = N // _BN
  return pl.pallas_call(
      _node1_body,
      grid=(grid,),
      in_specs=[
          pl.BlockSpec((2, _BN, EMB // 2), lambda i: (0, i, 0)),
          pl.BlockSpec((_BN, EMB), lambda i: (i, 0)),
          pl.BlockSpec(memory_space=pltpu.SMEM),
          pl.BlockSpec((EMB, EMB), lambda i: (0, 0)),
          pl.BlockSpec((1, EMB), lambda i: (0, 0)),
          pl.BlockSpec((EMB, EMB), lambda i: (0, 0)),
          pl.BlockSpec((1, EMB), lambda i: (0, 0)),
          pl.BlockSpec((EMB, PE), lambda i: (0, 0)),
          pl.BlockSpec((1, PE), lambda i: (0, 0)),
      ],
      out_specs=pl.BlockSpec((_BN, PE), lambda i: (i, 0)),
      out_shape=jax.ShapeDtypeStruct((N, PE), F32),
  )(h1S, agg, scale, W1, b1_row, W2, b2_row, out_W, ob_row)


def kernel(x, edge_index, edge_attr, eigvecs, eigvals, eps_param, phi_W1,
           phi_b1, phi_W2, phi_b2, edge_W, edge_b, g0_We, g0_be, g0_W1,
           g0_b1, g0_W2, g0_b2, g0_eps, g1_We, g1_be, g1_W1, g1_b1, g1_W2,
           g1_b2, g1_eps, out_W, out_b):
  src = edge_index[0]
  dst = edge_index[1]

  C0 = edge_W @ g0_We
  c0 = edge_b @ g0_We + g0_be
  C1 = edge_W @ g1_We
  c1 = edge_b @ g1_We + g1_be

  eaT = edge_attr.T
  b0L = jnp.concatenate([c0[:H // 2]] * 2).reshape(1, H)
  b0R = jnp.concatenate([c0[H // 2:]] * 2).reshape(1, H)
  b1L = jnp.concatenate([c1[:EMB // 2]] * 4).reshape(1, 2 * EMB)
  b1R = jnp.concatenate([c1[EMB // 2:]] * 4).reshape(1, 2 * EMB)

  hS = _eigen_stage(x, eigvecs, eigvals, eps_param.reshape(1, NV), phi_W1,
                    phi_b1.reshape(1, HID), phi_W2, phi_b2.reshape(1, H))
  e0P = _edge0_stage(eaT, C0, b0L, b0R)

  agg0 = _sc_segment(H)(hS, e0P.reshape(2, E, H // 2),
                        src.reshape(2, E // 2), dst.reshape(2, E // 2),
                        _destripe_table(H))
  e1P = _edge1_stage(eaT, C1, b1L, b1R)
  h1S = _node0_stage(hS, agg0, (1.0 + g0_eps).reshape(1, 1), g0_W1,
                     g0_b1.reshape(1, EMB), g0_W2, g0_b2.reshape(1, EMB))

  agg1 = _sc_segment(EMB)(h1S, e1P.reshape(2, E, EMB // 2),
                          src.reshape(4, E // 4), dst.reshape(4, E // 4),
                          _destripe_table(EMB))
  pe = _node1_stage(h1S, agg1, (1.0 + g1_eps).reshape(1, 1), g1_W1,
                    g1_b1.reshape(1, EMB), g1_W2, g1_b2.reshape(1, EMB),
                    out_W, out_b.reshape(1, PE))
  return pe


# speedup vs baseline: 1.0282x; 1.0282x over previous
import functools

import jax
import jax.numpy as jnp
import numpy as np
from jax import lax
from jax.experimental import pallas as pl
from jax.experimental.pallas import tpu as pltpu
from jax.experimental.pallas import tpu_sc as plsc

N = 10000
E = 320000
H = 128
NV = 16
DE = 16
EMB = 64
PE = 32
HID = 2 * H

F32 = jnp.float32


_BN = 1000


def _eigen_body(x_ref, vec_ref, val_ref, eps_ref, w1_ref, b1_ref, w2_ref,
                b2_ref, out_ref):
  ev = val_ref[...] + eps_ref[...]
  ev = jnp.where(jnp.isnan(ev), 0.0, ev)
  vec = jnp.where(jnp.isnan(vec_ref[...]), 0.0, vec_ref[...])
  w1 = w1_ref[...]
  b1 = b1_ref[...]
  w2 = w2_ref[...]
  b2 = b2_ref[...]
  acc = jnp.zeros((_BN, H), F32)
  for v in range(NV):
    t1 = vec[:, v:v + 1] * w1[0:1, :] + ev[:, v:v + 1] * w1[1:2, :] + b1
    t1 = jnp.maximum(t1, 0.0)
    t2 = jnp.dot(t1, w2, preferred_element_type=F32) + b2
    acc = acc + jnp.maximum(t2, 0.0)
  res = x_ref[...] + acc
  out_ref[0] = res[:, :H // 2]
  out_ref[1] = res[:, H // 2:]


def _eigen_stage(x, eigvecs, eigvals, eps_row, phi_W1, phi_b1, phi_W2,
                 phi_b2):
  grid = N // _BN
  return pl.pallas_call(
      _eigen_body,
      grid=(grid,),
      in_specs=[
          pl.BlockSpec((_BN, H), lambda i: (i, 0)),
          pl.BlockSpec((_BN, NV), lambda i: (i, 0)),
          pl.BlockSpec((_BN, NV), lambda i: (i, 0)),
          pl.BlockSpec((1, NV), lambda i: (0, 0)),
          pl.BlockSpec((2, HID), lambda i: (0, 0)),
          pl.BlockSpec((1, HID), lambda i: (0, 0)),
          pl.BlockSpec((HID, H), lambda i: (0, 0)),
          pl.BlockSpec((1, H), lambda i: (0, 0)),
      ],
      out_specs=pl.BlockSpec((2, _BN, H // 2), lambda i: (0, i, 0)),
      out_shape=jax.ShapeDtypeStruct((2, N, H // 2), F32),
  )(x, eigvecs, eigvals, eps_row, phi_W1, phi_b1, phi_W2, phi_b2)


_BE2 = 3200
_BE4 = 3200


def _dotT(a_ref, w_ref):
  return jax.lax.dot_general(a_ref[...], w_ref[...], (((0,), (0,)), ((), ())),
                             preferred_element_type=F32)


def _edge0_body(eaA_ref, eaB_ref, C0_ref, bL_ref, bR_ref, out_ref):
  EA = _dotT(eaA_ref, C0_ref)
  EB = _dotT(eaB_ref, C0_ref)
  hh = H // 2
  out_ref[0] = jnp.concatenate([EA[:, :hh], EB[:, :hh]], axis=1) + bL_ref[...]
  out_ref[1] = jnp.concatenate([EA[:, hh:], EB[:, hh:]], axis=1) + bR_ref[...]


def _edge0_stage(eaT, C0, bL, bR, half):
  grid = (E // 4) // _BE2
  nA = half * grid
  nB = (E // 2) // _BE2 + half * grid
  return pl.pallas_call(
      _edge0_body,
      grid=(grid,),
      in_specs=[
          pl.BlockSpec((DE, _BE2), lambda i: (0, i + nA)),
          pl.BlockSpec((DE, _BE2), lambda i: (0, i + nB)),
          pl.BlockSpec((DE, H), lambda i: (0, 0)),
          pl.BlockSpec((1, H), lambda i: (0, 0)),
          pl.BlockSpec((1, H), lambda i: (0, 0)),
      ],
      out_specs=pl.BlockSpec((2, _BE2, H), lambda i: (0, i, 0)),
      out_shape=jax.ShapeDtypeStruct((2, E // 4, H), F32),
  )(eaT, eaT, C0, bL, bR)


def _edge1_body(t0_ref, t1_ref, t2_ref, t3_ref, C1_ref, bL_ref, bR_ref,
                out_ref):
  F = [_dotT(t, C1_ref) for t in (t0_ref, t1_ref, t2_ref, t3_ref)]
  qq = EMB // 2
  out_ref[0] = jnp.concatenate([f[:, :qq] for f in F], axis=1) + bL_ref[...]
  out_ref[1] = jnp.concatenate([f[:, qq:] for f in F], axis=1) + bR_ref[...]


def _edge1_stage(eaT, C1, bL, bR):
  grid = (E // 4) // _BE4
  nb = grid
  return pl.pallas_call(
      _edge1_body,
      grid=(grid,),
      in_specs=[
          pl.BlockSpec((DE, _BE4), lambda i: (0, i)),
          pl.BlockSpec((DE, _BE4), lambda i: (0, i + nb)),
          pl.BlockSpec((DE, _BE4), lambda i: (0, i + 2 * nb)),
          pl.BlockSpec((DE, _BE4), lambda i: (0, i + 3 * nb)),
          pl.BlockSpec((DE, EMB), lambda i: (0, 0)),
          pl.BlockSpec((1, 2 * EMB), lambda i: (0, 0)),
          pl.BlockSpec((1, 2 * EMB), lambda i: (0, 0)),
      ],
      out_specs=pl.BlockSpec((2, _BE4, 2 * EMB), lambda i: (0, i, 0)),
      out_shape=jax.ShapeDtypeStruct((2, E // 4, 2 * EMB), F32),
  )(eaT, eaT, eaT, eaT, C1, bL, bR)


_NC = 2
_NS = 16
_G = 80
_NP = 10240


def _make_sc_segment(D, split=1, half=0):
  DH = D // 2
  P = 128 // DH
  C = 200 * P
  Q = C // _G
  EL = E // split
  per_tile = EL // _NS
  chunks = per_tile // C
  rp = _NP // _NS

  mesh = plsc.VectorSubcoreMesh(core_axis_name="c", subcore_axis_name="s",
                                num_cores=_NC, num_subcores=_NS)

  @functools.partial(
      pl.kernel,
      out_type=jax.ShapeDtypeStruct((_NP, D), F32),
      mesh=mesh,
      scratch_types=[
          pltpu.VMEM((2, P, C // P), jnp.int32),
          pltpu.VMEM((2, P, C // P), jnp.int32),
          pltpu.VMEM((2, Q * (_G // 16), 16), jnp.int32),
          pltpu.VMEM((2, Q, _G), jnp.int32),
          pltpu.VMEM((2, Q, _G), jnp.int32),
          pltpu.VMEM((C, DH), F32),
          pltpu.VMEM((C, DH), F32),
          pltpu.VMEM_SHARED((_NP, DH), F32),
          pltpu.SemaphoreType.DMA,
          pltpu.SemaphoreType.DMA,
          pltpu.SemaphoreType.DMA,
          pltpu.SemaphoreType.DMA,
          pltpu.SemaphoreType.DMA,
      ],
      compiler_params=pltpu.CompilerParams(use_tc_tiling_on_sc=False,
                                           needs_layout_passes=False),
  )
  def seg(h_hbm, e_hbm, srcP, dstP, itab_hbm, out_hbm, sraw, draw, itab,
          sidx, didx, bufA, bufB, agg, semLA, semLB, semGA, semGB, semS):
    c = lax.axis_index("c")
    s = lax.axis_index("s")
    bufs = (bufA, bufB)
    semL = (semLA, semLB)
    semG = (semGA, semGB)
    h_half = h_hbm.at[c]
    e_half = e_hbm.at[c]
    pltpu.sync_copy(itab_hbm, itab)

    def zrow(i, carry):
      for j in range(DH // 16):
        bufA[i, pl.ds(16 * j, 16)] = jnp.zeros((16,), F32)
      return carry

    lax.fori_loop(0, 128, zrow, 0)
    for q in range(rp // 128):
      pltpu.sync_copy(bufA.at[pl.ds(0, 128)],
                      agg.at[pl.ds(s * rp + q * 128, 128)])
    plsc.subcore_barrier()

    base_e = s * per_tile
    base_col = half * (EL // P) + s * (per_tile // P)

    def fire_ld(k, p):
      col = base_col + k * (C // P)
      pltpu.async_copy(srcP.at[:, pl.ds(col, C // P)], sraw.at[p], semL[p])
      pltpu.async_copy(dstP.at[:, pl.ds(col, C // P)], draw.at[p], semL[p])
      pltpu.async_copy(e_half.at[pl.ds(base_e + k * C, C)], bufs[p], semL[p])

    def wait_ld(k, p):
      col = base_col + k * (C // P)
      pltpu.make_async_copy(srcP.at[:, pl.ds(col, C // P)], sraw.at[p],
                            semL[p]).wait()
      pltpu.make_async_copy(dstP.at[:, pl.ds(col, C // P)], draw.at[p],
                            semL[p]).wait()
      pltpu.make_async_copy(e_half.at[pl.ds(base_e + k * C, C)], bufs[p],
                            semL[p]).wait()
      for q in range(Q):
        for v in range(_G // 16):
          j = q * (_G // 16) + v
          i0 = itab[0, j]
          i1 = itab[1, j]
          sidx[p, q, pl.ds(16 * v, 16)] = plsc.load_gather(
              sraw.at[p], [i0, i1])
          didx[p, q, pl.ds(16 * v, 16)] = plsc.load_gather(
              draw.at[p], [i0, i1])

    def fire_gathers(p):
      for q in range(Q):
        pltpu.async_copy(h_half.at[sidx.at[p, q]],
                         bufs[p].at[pl.ds(q * _G, _G)], semG[p], add=True)

    def drain_gathers(p):
      for q in range(Q):
        pltpu.make_async_copy(h_half.at[sidx.at[p, q]],
                              bufs[p].at[pl.ds(q * _G, _G)],
                              semG[p]).wait()

    def fire_scatter(p):
      for q in range(Q):
        pltpu.async_copy(bufs[p].at[pl.ds(q * _G, _G)],
                         agg.at[didx.at[p, q]], semS, add=True)

    def drain_scatter(p):
      for q in range(Q):
        pltpu.make_async_copy(bufs[p].at[pl.ds(q * _G, _G)],
                              agg.at[didx.at[p, q]], semS).wait()

    def relu(p):
      buf = bufs[p]

      def rrow(i, cc):
        for u in range(4):
          for j in range(DH // 16):
            sl = pl.ds(16 * j, 16)
            buf[4 * i + u, sl] = jnp.maximum(buf[4 * i + u, sl], 0.0)
        return cc

      lax.fori_loop(0, C // 4, rrow, 0)

    def process(k, p):
      drain_gathers(p)
      relu(p)
      fire_scatter(p)

      @pl.when(k + 1 < chunks)
      def _():
        wait_ld(k + 1, 1 - p)
        fire_gathers(1 - p)

      drain_scatter(p)

      @pl.when(k + 2 < chunks)
      def _():
        fire_ld(k + 2, p)

    fire_ld(0, 0)
    wait_ld(0, 0)
    fire_gathers(0)
    fire_ld(1, 1)

    def pair(i, carry):
      k = 2 * i
      process(k, 0)

      @pl.when(k + 1 < chunks)
      def _():
        process(k + 1, 1)

      return carry

    lax.fori_loop(0, (chunks + 1) // 2, pair, 0)

    plsc.subcore_barrier()
    pltpu.sync_copy(agg.at[pl.ds(s * rp, rp)],
                    out_hbm.at[pl.ds(s * rp, rp), pl.ds(c * DH, DH)])

  return seg


def _destripe_table(D):
  DH = D // 2
  P = 128 // DH
  C = 200 * P
  Q = C // _G
  lanes = np.arange(16, dtype=np.int32)
  i0 = []
  i1 = []
  for q in range(Q):
    for v in range(_G // 16):
      off = (16 * v + _G * q) // P
      i0.append(lanes % P)
      i1.append(lanes // P + off)
  return jnp.asarray(np.stack([np.stack(i0), np.stack(i1)]), jnp.int32)


_sc_cache = {}


def _sc_segment(D, split=1, half=0):
  key = (D, split, half)
  if key not in _sc_cache:
    _sc_cache[key] = _make_sc_segment(D, split, half)
  return _sc_cache[key]


def _node0_body(h_ref, agg_ref, aggb_ref, sc_ref, w1_ref, b1_ref, w2_ref,
                b2_ref, out_ref):
  hv = jnp.concatenate([h_ref[0], h_ref[1]], axis=1)
  hv = sc_ref[0, 0] * hv + agg_ref[...] + aggb_ref[...]
  t = jnp.maximum(jnp.dot(hv, w1_ref[...], preferred_element_type=F32)
                  + b1_ref[...], 0.0)
  res = jnp.maximum(
      jnp.dot(t, w2_ref[...], preferred_element_type=F32) + b2_ref[...], 0.0)
  out_ref[0] = res[:, :EMB // 2]
  out_ref[1] = res[:, EMB // 2:]


def _node0_stage(hS, agg, aggb, scale, W1, b1_row, W2, b2_row):
  grid = N // _BN
  return pl.pallas_call(
      _node0_body,
      grid=(grid,),
      in_specs=[
          pl.BlockSpec((2, _BN, H // 2), lambda i: (0, i, 0)),
          pl.BlockSpec((_BN, H), lambda i: (i, 0)),
          pl.BlockSpec((_BN, H), lambda i: (i, 0)),
          pl.BlockSpec(memory_space=pltpu.SMEM),
          pl.BlockSpec((H, EMB), lambda i: (0, 0)),
          pl.BlockSpec((1, EMB), lambda i: (0, 0)),
          pl.BlockSpec((EMB, EMB), lambda i: (0, 0)),
          pl.BlockSpec((1, EMB), lambda i: (0, 0)),
      ],
      out_specs=pl.BlockSpec((2, _BN, EMB // 2), lambda i: (0, i, 0)),
      out_shape=jax.ShapeDtypeStruct((2, N, EMB // 2), F32),
  )(hS, agg, aggb, scale, W1, b1_row, W2, b2_row)


def _node1_body(h_ref, agg_ref, sc_ref, w1_ref, b1_ref, w2_ref, b2_ref,
                ow_ref, ob_ref, out_ref):
  hv = jnp.concatenate([h_ref[0], h_ref[1]], axis=1)
  hv = sc_ref[0, 0] * hv + agg_ref[...]
  t = jnp.maximum(jnp.dot(hv, w1_ref[...], preferred_element_type=F32)
                  + b1_ref[...], 0.0)
  t = jnp.maximum(jnp.dot(t, w2_ref[...], preferred_element_type=F32)
                  + b2_ref[...], 0.0)
  out_ref[...] = jnp.dot(t, ow_ref[...], preferred_element_type=F32) + ob_ref[...]


def _node1_stage(h1S, agg, scale, W1, b1_row, W2, b2_row, out_W, ob_row):
  grid = N // _BN
  return pl.pallas_call(
      _node1_body,
      grid=(grid,),
      in_specs=[
          pl.BlockSpec((2, _BN, EMB // 2), lambda i: (0, i, 0)),
          pl.BlockSpec((_BN, EMB), lambda i: (i, 0)),
          pl.BlockSpec(memory_space=pltpu.SMEM),
          pl.BlockSpec((EMB, EMB), lambda i: (0, 0)),
          pl.BlockSpec((1, EMB), lambda i: (0, 0)),
          pl.BlockSpec((EMB, EMB), lambda i: (0, 0)),
          pl.BlockSpec((1, EMB), lambda i: (0, 0)),
          pl.BlockSpec((EMB, PE), lambda i: (0, 0)),
          pl.BlockSpec((1, PE), lambda i: (0, 0)),
      ],
      out_specs=pl.BlockSpec((_BN, PE), lambda i: (i, 0)),
      out_shape=jax.ShapeDtypeStruct((N, PE), F32),
  )(h1S, agg, scale, W1, b1_row, W2, b2_row, out_W, ob_row)


def kernel(x, edge_index, edge_attr, eigvecs, eigvals, eps_param, phi_W1,
           phi_b1, phi_W2, phi_b2, edge_W, edge_b, g0_We, g0_be, g0_W1,
           g0_b1, g0_W2, g0_b2, g0_eps, g1_We, g1_be, g1_W1, g1_b1, g1_W2,
           g1_b2, g1_eps, out_W, out_b):
  src = edge_index[0]
  dst = edge_index[1]

  C0 = edge_W @ g0_We
  c0 = edge_b @ g0_We + g0_be
  C1 = edge_W @ g1_We
  c1 = edge_b @ g1_We + g1_be

  eaT = edge_attr.T
  b0L = jnp.concatenate([c0[:H // 2]] * 2).reshape(1, H)
  b0R = jnp.concatenate([c0[H // 2:]] * 2).reshape(1, H)
  b1L = jnp.concatenate([c1[:EMB // 2]] * 4).reshape(1, 2 * EMB)
  b1R = jnp.concatenate([c1[EMB // 2:]] * 4).reshape(1, 2 * EMB)

  hS = _eigen_stage(x, eigvecs, eigvals, eps_param.reshape(1, NV), phi_W1,
                    phi_b1.reshape(1, HID), phi_W2, phi_b2.reshape(1, H))
  srcP2 = src.reshape(2, E // 2)
  dstP2 = dst.reshape(2, E // 2)
  itab0 = _destripe_table(H)
  e0Pa = _edge0_stage(eaT, C0, b0L, b0R, 0)
  agg0a = _sc_segment(H, 2, 0)(hS, e0Pa.reshape(2, E // 2, H // 2),
                               srcP2, dstP2, itab0)
  e0Pb = _edge0_stage(eaT, C0, b0L, b0R, 1)
  agg0b = _sc_segment(H, 2, 1)(hS, e0Pb.reshape(2, E // 2, H // 2),
                               srcP2, dstP2, itab0)
  e1P = _edge1_stage(eaT, C1, b1L, b1R)
  h1S = _node0_stage(hS, agg0a, agg0b, (1.0 + g0_eps).reshape(1, 1), g0_W1,
                     g0_b1.reshape(1, EMB), g0_W2, g0_b2.reshape(1, EMB))

  agg1 = _sc_segment(EMB)(h1S, e1P.reshape(2, E, EMB // 2),
                          src.reshape(4, E // 4), dst.reshape(4, E // 4),
                          _destripe_table(EMB))
  pe = _node1_stage(h1S, agg1, (1.0 + g1_eps).reshape(1, 1), g1_W1,
                    g1_b1.reshape(1, EMB), g1_W2, g1_b2.reshape(1, EMB),
                    out_W, out_b.reshape(1, PE))
  return pe


# speedup vs baseline: 1.0287x; 1.0005x over previous
import functools

import jax
import jax.numpy as jnp
import numpy as np
from jax import lax
from jax.experimental import pallas as pl
from jax.experimental.pallas import tpu as pltpu
from jax.experimental.pallas import tpu_sc as plsc

N = 10000
E = 320000
H = 128
NV = 16
DE = 16
EMB = 64
PE = 32
HID = 2 * H

F32 = jnp.float32


_BN = 1000


def _eigen_body(x_ref, vec_ref, val_ref, eps_ref, w1_ref, b1_ref, w2_ref,
                b2_ref, out_ref):
  BF = jnp.bfloat16
  ev = val_ref[...] + eps_ref[...]
  ev = jnp.where(jnp.isnan(ev), 0.0, ev)
  vec = jnp.where(jnp.isnan(vec_ref[...]), 0.0, vec_ref[...])
  vecb = vec.astype(BF)
  evb = ev.astype(BF)
  w1b = w1_ref[...].astype(BF)
  b1 = b1_ref[...]
  w2b = w2_ref[...].astype(BF)
  b2 = b2_ref[...]
  acc = jnp.zeros((_BN, H), F32)
  for v in range(NV):
    sv = jnp.concatenate([vecb[:, v:v + 1], evb[:, v:v + 1]], axis=1)
    t1 = jnp.dot(sv, w1b, preferred_element_type=F32) + b1
    t1 = jnp.maximum(t1, 0.0).astype(BF)
    t2 = jnp.dot(t1, w2b, preferred_element_type=F32) + b2
    acc = acc + jnp.maximum(t2, 0.0)
  res = x_ref[...] + acc
  out_ref[0] = res[:, :H // 2]
  out_ref[1] = res[:, H // 2:]


def _eigen_stage(x, eigvecs, eigvals, eps_row, phi_W1, phi_b1, phi_W2,
                 phi_b2):
  grid = N // _BN
  return pl.pallas_call(
      _eigen_body,
      grid=(grid,),
      in_specs=[
          pl.BlockSpec((_BN, H), lambda i: (i, 0)),
          pl.BlockSpec((_BN, NV), lambda i: (i, 0)),
          pl.BlockSpec((_BN, NV), lambda i: (i, 0)),
          pl.BlockSpec((1, NV), lambda i: (0, 0)),
          pl.BlockSpec((2, HID), lambda i: (0, 0)),
          pl.BlockSpec((1, HID), lambda i: (0, 0)),
          pl.BlockSpec((HID, H), lambda i: (0, 0)),
          pl.BlockSpec((1, H), lambda i: (0, 0)),
      ],
      out_specs=pl.BlockSpec((2, _BN, H // 2), lambda i: (0, i, 0)),
      out_shape=jax.ShapeDtypeStruct((2, N, H // 2), F32),
  )(x, eigvecs, eigvals, eps_row, phi_W1, phi_b1, phi_W2, phi_b2)


_BE2 = 3200
_BE4 = 3200


def _dotT(a_ref, w_ref):
  return jax.lax.dot_general(a_ref[...], w_ref[...], (((0,), (0,)), ((), ())),
                             preferred_element_type=F32)


def _edge0_body(eaA_ref, eaB_ref, C0_ref, bL_ref, bR_ref, out_ref):
  EA = _dotT(eaA_ref, C0_ref)
  EB = _dotT(eaB_ref, C0_ref)
  hh = H // 2
  out_ref[0] = jnp.concatenate([EA[:, :hh], EB[:, :hh]], axis=1) + bL_ref[...]
  out_ref[1] = jnp.concatenate([EA[:, hh:], EB[:, hh:]], axis=1) + bR_ref[...]


def _edge0_stage(eaT, C0, bL, bR, half):
  grid = (E // 4) // _BE2
  nA = half * grid
  nB = (E // 2) // _BE2 + half * grid
  return pl.pallas_call(
      _edge0_body,
      grid=(grid,),
      in_specs=[
          pl.BlockSpec((DE, _BE2), lambda i: (0, i + nA)),
          pl.BlockSpec((DE, _BE2), lambda i: (0, i + nB)),
          pl.BlockSpec((DE, H), lambda i: (0, 0)),
          pl.BlockSpec((1, H), lambda i: (0, 0)),
          pl.BlockSpec((1, H), lambda i: (0, 0)),
      ],
      out_specs=pl.BlockSpec((2, _BE2, H), lambda i: (0, i, 0)),
      out_shape=jax.ShapeDtypeStruct((2, E // 4, H), F32),
  )(eaT, eaT, C0, bL, bR)


def _edge1_body(t0_ref, t1_ref, t2_ref, t3_ref, C1_ref, bL_ref, bR_ref,
                out_ref):
  F = [_dotT(t, C1_ref) for t in (t0_ref, t1_ref, t2_ref, t3_ref)]
  qq = EMB // 2
  out_ref[0] = jnp.concatenate([f[:, :qq] for f in F], axis=1) + bL_ref[...]
  out_ref[1] = jnp.concatenate([f[:, qq:] for f in F], axis=1) + bR_ref[...]


def _edge1_stage(eaT, C1, bL, bR):
  grid = (E // 4) // _BE4
  nb = grid
  return pl.pallas_call(
      _edge1_body,
      grid=(grid,),
      in_specs=[
          pl.BlockSpec((DE, _BE4), lambda i: (0, i)),
          pl.BlockSpec((DE, _BE4), lambda i: (0, i + nb)),
          pl.BlockSpec((DE, _BE4), lambda i: (0, i + 2 * nb)),
          pl.BlockSpec((DE, _BE4), lambda i: (0, i + 3 * nb)),
          pl.BlockSpec((DE, EMB), lambda i: (0, 0)),
          pl.BlockSpec((1, 2 * EMB), lambda i: (0, 0)),
          pl.BlockSpec((1, 2 * EMB), lambda i: (0, 0)),
      ],
      out_specs=pl.BlockSpec((2, _BE4, 2 * EMB), lambda i: (0, i, 0)),
      out_shape=jax.ShapeDtypeStruct((2, E // 4, 2 * EMB), F32),
  )(eaT, eaT, eaT, eaT, C1, bL, bR)


_NC = 2
_NS = 16
_G = 80
_NP = 10240


def _make_sc_segment(D, split=1, half=0):
  DH = D // 2
  P = 128 // DH
  C = 200 * P
  Q = C // _G
  EL = E // split
  per_tile = EL // _NS
  chunks = per_tile // C
  rp = _NP // _NS

  mesh = plsc.VectorSubcoreMesh(core_axis_name="c", subcore_axis_name="s",
                                num_cores=_NC, num_subcores=_NS)

  @functools.partial(
      pl.kernel,
      out_type=jax.ShapeDtypeStruct((_NP, D), F32),
      mesh=mesh,
      scratch_types=[
          pltpu.VMEM((2, P, C // P), jnp.int32),
          pltpu.VMEM((2, P, C // P), jnp.int32),
          pltpu.VMEM((2, Q * (_G // 16), 16), jnp.int32),
          pltpu.VMEM((2, Q, _G), jnp.int32),
          pltpu.VMEM((2, Q, _G), jnp.int32),
          pltpu.VMEM((C, DH), F32),
          pltpu.VMEM((C, DH), F32),
          pltpu.VMEM_SHARED((_NP, DH), F32),
          pltpu.SemaphoreType.DMA,
          pltpu.SemaphoreType.DMA,
          pltpu.SemaphoreType.DMA,
          pltpu.SemaphoreType.DMA,
          pltpu.SemaphoreType.DMA,
      ],
      compiler_params=pltpu.CompilerParams(use_tc_tiling_on_sc=False,
                                           needs_layout_passes=False),
  )
  def seg(h_hbm, e_hbm, srcP, dstP, itab_hbm, out_hbm, sraw, draw, itab,
          sidx, didx, bufA, bufB, agg, semLA, semLB, semGA, semGB, semS):
    c = lax.axis_index("c")
    s = lax.axis_index("s")
    bufs = (bufA, bufB)
    semL = (semLA, semLB)
    semG = (semGA, semGB)
    h_half = h_hbm.at[c]
    e_half = e_hbm.at[c]
    pltpu.sync_copy(itab_hbm, itab)

    def zrow(i, carry):
      for j in range(DH // 16):
        bufA[i, pl.ds(16 * j, 16)] = jnp.zeros((16,), F32)
      return carry

    lax.fori_loop(0, 128, zrow, 0)
    for q in range(rp // 128):
      pltpu.sync_copy(bufA.at[pl.ds(0, 128)],
                      agg.at[pl.ds(s * rp + q * 128, 128)])
    plsc.subcore_barrier()

    base_e = s * per_tile
    base_col = half * (EL // P) + s * (per_tile // P)

    def fire_ld(k, p):
      col = base_col + k * (C // P)
      pltpu.async_copy(srcP.at[:, pl.ds(col, C // P)], sraw.at[p], semL[p])
      pltpu.async_copy(dstP.at[:, pl.ds(col, C // P)], draw.at[p], semL[p])
      pltpu.async_copy(e_half.at[pl.ds(base_e + k * C, C)], bufs[p], semL[p])

    def wait_ld(k, p):
      col = base_col + k * (C // P)
      pltpu.make_async_copy(srcP.at[:, pl.ds(col, C // P)], sraw.at[p],
                            semL[p]).wait()
      pltpu.make_async_copy(dstP.at[:, pl.ds(col, C // P)], draw.at[p],
                            semL[p]).wait()
      pltpu.make_async_copy(e_half.at[pl.ds(base_e + k * C, C)], bufs[p],
                            semL[p]).wait()
      for q in range(Q):
        for v in range(_G // 16):
          j = q * (_G // 16) + v
          i0 = itab[0, j]
          i1 = itab[1, j]
          sidx[p, q, pl.ds(16 * v, 16)] = plsc.load_gather(
              sraw.at[p], [i0, i1])
          didx[p, q, pl.ds(16 * v, 16)] = plsc.load_gather(
              draw.at[p], [i0, i1])

    def fire_gathers(p):
      for q in range(Q):
        pltpu.async_copy(h_half.at[sidx.at[p, q]],
                         bufs[p].at[pl.ds(q * _G, _G)], semG[p], add=True)

    def drain_gathers(p):
      for q in range(Q):
        pltpu.make_async_copy(h_half.at[sidx.at[p, q]],
                              bufs[p].at[pl.ds(q * _G, _G)],
                              semG[p]).wait()

    def fire_scatter(p):
      for q in range(Q):
        pltpu.async_copy(bufs[p].at[pl.ds(q * _G, _G)],
                         agg.at[didx.at[p, q]], semS, add=True)

    def drain_scatter(p):
      for q in range(Q):
        pltpu.make_async_copy(bufs[p].at[pl.ds(q * _G, _G)],
                              agg.at[didx.at[p, q]], semS).wait()

    def relu(p):
      buf = bufs[p]

      def rrow(i, cc):
        for u in range(4):
          for j in range(DH // 16):
            sl = pl.ds(16 * j, 16)
            buf[4 * i + u, sl] = jnp.maximum(buf[4 * i + u, sl], 0.0)
        return cc

      lax.fori_loop(0, C // 4, rrow, 0)

    def process(k, p):
      drain_gathers(p)
      relu(p)
      fire_scatter(p)

      @pl.when(k + 1 < chunks)
      def _():
        wait_ld(k + 1, 1 - p)
        fire_gathers(1 - p)

      drain_scatter(p)

      @pl.when(k + 2 < chunks)
      def _():
        fire_ld(k + 2, p)

    fire_ld(0, 0)
    wait_ld(0, 0)
    fire_gathers(0)
    fire_ld(1, 1)

    def pair(i, carry):
      k = 2 * i
      process(k, 0)

      @pl.when(k + 1 < chunks)
      def _():
        process(k + 1, 1)

      return carry

    lax.fori_loop(0, (chunks + 1) // 2, pair, 0)

    plsc.subcore_barrier()
    pltpu.sync_copy(agg.at[pl.ds(s * rp, rp)],
                    out_hbm.at[pl.ds(s * rp, rp), pl.ds(c * DH, DH)])

  return seg


def _destripe_table(D):
  DH = D // 2
  P = 128 // DH
  C = 200 * P
  Q = C // _G
  lanes = np.arange(16, dtype=np.int32)
  i0 = []
  i1 = []
  for q in range(Q):
    for v in range(_G // 16):
      off = (16 * v + _G * q) // P
      i0.append(lanes % P)
      i1.append(lanes // P + off)
  return jnp.asarray(np.stack([np.stack(i0), np.stack(i1)]), jnp.int32)


_sc_cache = {}


def _sc_segment(D, split=1, half=0):
  key = (D, split, half)
  if key not in _sc_cache:
    _sc_cache[key] = _make_sc_segment(D, split, half)
  return _sc_cache[key]


def _node0_body(h_ref, agg_ref, aggb_ref, sc_ref, w1_ref, b1_ref, w2_ref,
                b2_ref, out_ref):
  hv = jnp.concatenate([h_ref[0], h_ref[1]], axis=1)
  hv = sc_ref[0, 0] * hv + agg_ref[...] + aggb_ref[...]
  t = jnp.maximum(jnp.dot(hv, w1_ref[...], preferred_element_type=F32)
                  + b1_ref[...], 0.0)
  res = jnp.maximum(
      jnp.dot(t, w2_ref[...], preferred_element_type=F32) + b2_ref[...], 0.0)
  out_ref[0] = res[:, :EMB // 2]
  out_ref[1] = res[:, EMB // 2:]


def _node0_stage(hS, agg, aggb, scale, W1, b1_row, W2, b2_row):
  grid = N // _BN
  return pl.pallas_call(
      _node0_body,
      grid=(grid,),
      in_specs=[
          pl.BlockSpec((2, _BN, H // 2), lambda i: (0, i, 0)),
          pl.BlockSpec((_BN, H), lambda i: (i, 0)),
          pl.BlockSpec((_BN, H), lambda i: (i, 0)),
          pl.BlockSpec(memory_space=pltpu.SMEM),
          pl.BlockSpec((H, EMB), lambda i: (0, 0)),
          pl.BlockSpec((1, EMB), lambda i: (0, 0)),
          pl.BlockSpec((EMB, EMB), lambda i: (0, 0)),
          pl.BlockSpec((1, EMB), lambda i: (0, 0)),
      ],
      out_specs=pl.BlockSpec((2, _BN, EMB // 2), lambda i: (0, i, 0)),
      out_shape=jax.ShapeDtypeStruct((2, N, EMB // 2), F32),
  )(hS, agg, aggb, scale, W1, b1_row, W2, b2_row)


def _node1_body(h_ref, agg_ref, sc_ref, w1_ref, b1_ref, w2_ref, b2_ref,
                ow_ref, ob_ref, out_ref):
  hv = jnp.concatenate([h_ref[0], h_ref[1]], axis=1)
  hv = sc_ref[0, 0] * hv + agg_ref[...]
  t = jnp.maximum(jnp.dot(hv, w1_ref[...], preferred_element_type=F32)
                  + b1_ref[...], 0.0)
  t = jnp.maximum(jnp.dot(t, w2_ref[...], preferred_element_type=F32)
                  + b2_ref[...], 0.0)
  out_ref[...] = jnp.dot(t, ow_ref[...], preferred_element_type=F32) + ob_ref[...]


def _node1_stage(h1S, agg, scale, W1, b1_row, W2, b2_row, out_W, ob_row):
  grid = N // _BN
  return pl.pallas_call(
      _node1_body,
      grid=(grid,),
      in_specs=[
          pl.BlockSpec((2, _BN, EMB // 2), lambda i: (0, i, 0)),
          pl.BlockSpec((_BN, EMB), lambda i: (i, 0)),
          pl.BlockSpec(memory_space=pltpu.SMEM),
          pl.BlockSpec((EMB, EMB), lambda i: (0, 0)),
          pl.BlockSpec((1, EMB), lambda i: (0, 0)),
          pl.BlockSpec((EMB, EMB), lambda i: (0, 0)),
          pl.BlockSpec((1, EMB), lambda i: (0, 0)),
          pl.BlockSpec((EMB, PE), lambda i: (0, 0)),
          pl.BlockSpec((1, PE), lambda i: (0, 0)),
      ],
      out_specs=pl.BlockSpec((_BN, PE), lambda i: (i, 0)),
      out_shape=jax.ShapeDtypeStruct((N, PE), F32),
  )(h1S, agg, scale, W1, b1_row, W2, b2_row, out_W, ob_row)


def kernel(x, edge_index, edge_attr, eigvecs, eigvals, eps_param, phi_W1,
           phi_b1, phi_W2, phi_b2, edge_W, edge_b, g0_We, g0_be, g0_W1,
           g0_b1, g0_W2, g0_b2, g0_eps, g1_We, g1_be, g1_W1, g1_b1, g1_W2,
           g1_b2, g1_eps, out_W, out_b):
  src = edge_index[0]
  dst = edge_index[1]

  C0 = edge_W @ g0_We
  c0 = edge_b @ g0_We + g0_be
  C1 = edge_W @ g1_We
  c1 = edge_b @ g1_We + g1_be

  eaT = edge_attr.T
  b0L = jnp.concatenate([c0[:H // 2]] * 2).reshape(1, H)
  b0R = jnp.concatenate([c0[H // 2:]] * 2).reshape(1, H)
  b1L = jnp.concatenate([c1[:EMB // 2]] * 4).reshape(1, 2 * EMB)
  b1R = jnp.concatenate([c1[EMB // 2:]] * 4).reshape(1, 2 * EMB)

  hS = _eigen_stage(x, eigvecs, eigvals, eps_param.reshape(1, NV), phi_W1,
                    phi_b1.reshape(1, HID), phi_W2, phi_b2.reshape(1, H))
  srcP2 = src.reshape(2, E // 2)
  dstP2 = dst.reshape(2, E // 2)
  itab0 = _destripe_table(H)
  e0Pa = _edge0_stage(eaT, C0, b0L, b0R, 0)
  agg0a = _sc_segment(H, 2, 0)(hS, e0Pa.reshape(2, E // 2, H // 2),
                               srcP2, dstP2, itab0)
  e0Pb = _edge0_stage(eaT, C0, b0L, b0R, 1)
  agg0b = _sc_segment(H, 2, 1)(hS, e0Pb.reshape(2, E // 2, H // 2),
                               srcP2, dstP2, itab0)
  e1P = _edge1_stage(eaT, C1, b1L, b1R)
  h1S = _node0_stage(hS, agg0a, agg0b, (1.0 + g0_eps).reshape(1, 1), g0_W1,
                     g0_b1.reshape(1, EMB), g0_W2, g0_b2.reshape(1, EMB))

  agg1 = _sc_segment(EMB)(h1S, e1P.reshape(2, E, EMB // 2),
                          src.reshape(4, E // 4), dst.reshape(4, E // 4),
                          _destripe_table(EMB))
  pe = _node1_stage(h1S, agg1, (1.0 + g1_eps).reshape(1, 1), g1_W1,
                    g1_b1.reshape(1, EMB), g1_W2, g1_b2.reshape(1, EMB),
                    out_W, out_b.reshape(1, PE))
  return pe
